# baseline (speedup 1.0000x reference)
import functools

import jax
import jax.numpy as jnp
from jax import lax
from jax.experimental import pallas as pl
from jax.experimental.pallas import tpu as pltpu
from jax.experimental.pallas import tpu_sc as plsc

VOCAB = 1000000
EMBED = 64
BATCH = 16384
PAIRW = 2 * EMBED

NUM_CORES = 2
NUM_SUBCORES = 16
LANES = 16
NW = NUM_CORES * NUM_SUBCORES
B_PER_W = BATCH // NW
CHUNK = 256
N_CHUNKS = B_PER_W // CHUNK
GROUPS = CHUNK // LANES


def _make_sc_kernel():
    mesh = plsc.VectorSubcoreMesh(core_axis_name="c", subcore_axis_name="s")

    @functools.partial(
        pl.kernel,
        mesh=mesh,
        out_type=jax.ShapeDtypeStruct((BATCH,), jnp.float32),
        compiler_params=pltpu.CompilerParams(needs_layout_passes=False),
        scratch_types=[
            pltpu.VMEM((B_PER_W,), jnp.int32),
            pltpu.VMEM((B_PER_W,), jnp.int32),
            pltpu.VMEM((B_PER_W,), jnp.int32),
            pltpu.VMEM((B_PER_W,), jnp.int32),
            pltpu.VMEM((CHUNK, PAIRW), jnp.float32),
            pltpu.VMEM((CHUNK, PAIRW), jnp.float32),
            pltpu.VMEM((B_PER_W,), jnp.float32),
            pltpu.SemaphoreType.DMA,
            pltpu.SemaphoreType.DMA,
        ],
    )
    def k(idx0_hbm, idx1_hbm, emb1_hbm, emb2_hbm, out_hbm,
          idx0_v, idx1_v, pair0_v, pair1_v, u_v, v_v, out_v, sem0, sem1):
        wid = lax.axis_index("s") * NUM_CORES + lax.axis_index("c")
        base = wid * B_PER_W

        pltpu.sync_copy(idx0_hbm.at[pl.ds(base, B_PER_W)], idx0_v)
        pltpu.sync_copy(idx1_hbm.at[pl.ds(base, B_PER_W)], idx1_v)
        for i in range(B_PER_W // LANES):
            sl = pl.ds(i * LANES, LANES)
            pair0_v[sl] = idx0_v[sl] >> 1
            pair1_v[sl] = idx1_v[sl] >> 1

        lane = lax.iota(jnp.int32, LANES)

        for c in range(N_CHUNKS):
            cp0 = pltpu.async_copy(
                emb1_hbm.at[pair0_v.at[pl.ds(c * CHUNK, CHUNK)]], u_v, sem0)
            cp1 = pltpu.async_copy(
                emb2_hbm.at[pair1_v.at[pl.ds(c * CHUNK, CHUNK)]], v_v, sem1)
            cp0.wait()
            cp1.wait()

            def group(g, carry):
                rows = g * LANES + lane
                half0 = (idx0_v[pl.ds(c * CHUNK + g * LANES, LANES)] & 1) << 6
                half1 = (idx1_v[pl.ds(c * CHUNK + g * LANES, LANES)] & 1) << 6
                acc = jnp.zeros((LANES,), jnp.float32)
                for d in range(EMBED):
                    u = plsc.load_gather(u_v, [rows, half0 + d])
                    v = plsc.load_gather(v_v, [rows, half1 + d])
                    acc = acc + u * v
                out_v[pl.ds(c * CHUNK + g * LANES, LANES)] = (
                    1.0 / (1.0 + jnp.exp(-acc)))
                return carry

            lax.fori_loop(0, GROUPS, group, 0)

        pltpu.sync_copy(out_v, out_hbm.at[pl.ds(base, B_PER_W)])

    return k


_sc_kernel = _make_sc_kernel()


@jax.jit
def kernel(X_batch, emb1, emb2):
    idx0 = X_batch[:, 0].astype(jnp.int32)
    idx1 = X_batch[:, 1].astype(jnp.int32)
    emb1p = jnp.reshape(emb1, (VOCAB // 2, PAIRW))
    emb2p = jnp.reshape(emb2, (VOCAB // 2, PAIRW))
    return _sc_kernel(idx0, idx1, emb1p, emb2p)


# speedup vs baseline: 3.3232x; 3.3232x over previous
import functools

import jax
import jax.numpy as jnp
from jax import lax
from jax.experimental import pallas as pl
from jax.experimental.pallas import tpu as pltpu
from jax.experimental.pallas import tpu_sc as plsc

VOCAB = 1000000
EMBED = 64
BATCH = 16384
BLK = 128

NUM_CORES = 2
NUM_SUBCORES = 16
LANES = 16
NW = NUM_CORES * NUM_SUBCORES
B_PER_W = BATCH // NW
NQ = EMBED // LANES
DEPTH = 4
LOOKAHEAD = 3


def _make_phase1():
    mesh = plsc.VectorSubcoreMesh(core_axis_name="c", subcore_axis_name="s")

    @functools.partial(
        pl.kernel,
        mesh=mesh,
        out_type=(
            jax.ShapeDtypeStruct((BATCH, BLK), jnp.float32),
            jax.ShapeDtypeStruct((BATCH, BLK), jnp.float32),
        ),
        compiler_params=pltpu.CompilerParams(needs_layout_passes=False),
        scratch_types=[
            pltpu.VMEM((B_PER_W,), jnp.int32),
            pltpu.VMEM((B_PER_W // LANES, LANES), jnp.int32),
            pltpu.VMEM((DEPTH, EMBED, BLK), jnp.float32),
            pltpu.VMEM((2, LANES, BLK), jnp.float32),
            pltpu.SemaphoreType.DMA,
            pltpu.SemaphoreType.DMA,
        ],
    )
    def k(s0_hbm, p0_hbm, s1_hbm, p1_hbm, e1t_hbm, e2t_hbm,
          u_hbm, v_hbm, sidx_v, pv2, ring, rowstage, sem_b, sem_s):
        wid = lax.axis_index("s") * NUM_CORES + lax.axis_index("c")
        base = wid * B_PER_W
        lane = lax.iota(jnp.int32, LANES)

        def ext(vec, j):
            return jnp.max(jnp.where(lane == j, vec, 0))

        for s_hbm, p_hbm, t_hbm, o_hbm in (
                (s0_hbm, p0_hbm, e1t_hbm, u_hbm),
                (s1_hbm, p1_hbm, e2t_hbm, v_hbm)):
            pltpu.sync_copy(s_hbm.at[pl.ds(base, B_PER_W)], sidx_v)
            pltpu.sync_copy(
                p_hbm.at[pl.ds(wid * (B_PER_W // LANES), B_PER_W // LANES)],
                pv2)

            def fire(col, slot):
                pltpu.async_copy(
                    t_hbm.at[:, pl.ds(pl.multiple_of(col, BLK), BLK)],
                    ring.at[slot], sem_b)

            def drain_block():
                pltpu.make_async_copy(
                    t_hbm.at[:, pl.ds(0, BLK)], ring.at[0], sem_b).wait()

            def drain_scatter():
                pltpu.make_async_copy(
                    o_hbm.at[pl.ds(0, LANES)], rowstage.at[0], sem_s).wait()

            cur0 = sidx_v[pl.ds(0, LANES)]
            prv0 = plsc.load_gather(sidx_v, [jnp.maximum(lane - 1, 0)])
            new0 = (((cur0 >> 7) != (prv0 >> 7)) | (lane == 0)) & (
                lane < LOOKAHEAD)
            inc0 = plsc.cumsum(new0.astype(jnp.int32))
            pk0 = (cur0 >> 7) | (inc0 << 13)
            jf_prev = jnp.int32(0)
            for j in range(LOOKAHEAD):
                pj = ext(pk0, j)
                jf_j = pj >> 13

                @pl.when(jf_j != jf_prev)
                def _(pj=pj, jf_j=jf_j):
                    fire((pj & 8191) << 7, (jf_j - 1) & (DEPTH - 1))

                jf_prev = jf_j
            jf0 = jnp.max(inc0)

            def body(g, carry):
                jf_base, ju_base = carry
                gpos = g * LANES + lane
                gvec = plsc.load_gather(sidx_v, [gpos])
                prv = plsc.load_gather(sidx_v, [jnp.maximum(gpos - 1, 0)])
                newc = ((gvec >> 7) != (prv >> 7)) | (gpos == 0)
                ju_vec = ju_base + plsc.cumsum(newc.astype(jnp.int32))

                fpos = gpos + LOOKAHEAD
                fval = fpos < B_PER_W
                fposc = jnp.minimum(fpos, B_PER_W - 1)
                fvec = plsc.load_gather(sidx_v, [fposc])
                fprv = plsc.load_gather(sidx_v, [fposc - 1])
                newf = ((fvec >> 7) != (fprv >> 7)) & fval
                jf_vec = jf_base + plsc.cumsum(newf.astype(jnp.int32))

                pka = (gvec & (BLK - 1)) | (ju_vec << 7)
                pkb = (fvec >> 7) | (jf_vec << 13)

                gb = g & 1
                ju_prev = ju_base
                jf_prev = jf_base
                for j in range(LANES):
                    pb = ext(pkb, j)
                    jf_j = pb >> 13

                    @pl.when(jf_j != jf_prev)
                    def _(pb=pb, jf_j=jf_j):
                        fire((pb & 8191) << 7, (jf_j - 1) & (DEPTH - 1))

                    jf_prev = jf_j

                    pa = ext(pka, j)
                    ju_j = pa >> 7

                    @pl.when(ju_j != ju_prev)
                    def _():
                        drain_block()

                    ju_prev = ju_j
                    bi = (ju_j - 1) & (DEPTH - 1)
                    cols = jnp.full((LANES,), pa & (BLK - 1), jnp.int32)
                    for q in range(NQ):
                        rows = q * LANES + lane
                        rowstage[gb, j, pl.ds(q * LANES, LANES)] = (
                            plsc.load_gather(ring.at[bi], [rows, cols]))

                pltpu.async_copy(rowstage.at[gb], o_hbm.at[pv2.at[g]], sem_s)

                @pl.when(g >= 1)
                def _():
                    drain_scatter()

                return jf_prev, ju_prev

            lax.fori_loop(0, B_PER_W // LANES, body, (jf0, jnp.int32(0)))
            drain_scatter()

    return k


def _make_phase2():
    mesh = plsc.VectorSubcoreMesh(core_axis_name="c", subcore_axis_name="s")

    @functools.partial(
        pl.kernel,
        mesh=mesh,
        out_type=jax.ShapeDtypeStruct((BATCH,), jnp.float32),
        compiler_params=pltpu.CompilerParams(needs_layout_passes=False),
        scratch_types=[
            pltpu.VMEM((B_PER_W // 2, BLK), jnp.float32),
            pltpu.VMEM((B_PER_W // 2, BLK), jnp.float32),
            pltpu.VMEM((B_PER_W,), jnp.float32),
            pltpu.SemaphoreType.DMA,
            pltpu.SemaphoreType.DMA,
        ],
    )
    def k(u_hbm, v_hbm, out_hbm, u_v, v_v, out_v, sem0, sem1):
        wid = lax.axis_index("s") * NUM_CORES + lax.axis_index("c")
        base = wid * B_PER_W
        half = B_PER_W // 2

        lane = lax.iota(jnp.int32, LANES)
        cols = [jnp.full((LANES,), d, jnp.int32) for d in range(EMBED)]

        for c in range(2):
            cp0 = pltpu.async_copy(
                u_hbm.at[pl.ds(base + c * half, half)], u_v, sem0)
            cp1 = pltpu.async_copy(
                v_hbm.at[pl.ds(base + c * half, half)], v_v, sem1)
            cp0.wait()
            cp1.wait()

            def group(g, carry):
                rows = g * LANES + lane
                acc = jnp.zeros((LANES,), jnp.float32)
                for d in range(EMBED):
                    u = plsc.load_gather(u_v, [rows, cols[d]])
                    v = plsc.load_gather(v_v, [rows, cols[d]])
                    acc = acc + u * v
                out_v[pl.ds(c * half + g * LANES, LANES)] = (
                    1.0 / (1.0 + jnp.exp(-acc)))
                return carry

            lax.fori_loop(0, half // LANES, group, 0)

        pltpu.sync_copy(out_v, out_hbm.at[pl.ds(base, B_PER_W)])

    return k


_phase1 = _make_phase1()
_phase2 = _make_phase2()


@jax.jit
def kernel(X_batch, emb1, emb2):
    idx0 = X_batch[:, 0].astype(jnp.int32)
    idx1 = X_batch[:, 1].astype(jnp.int32)
    pos = lax.iota(jnp.int32, BATCH)
    s0, p0 = lax.sort((idx0, pos), num_keys=1)
    s1, p1 = lax.sort((idx1, pos), num_keys=1)
    p0r = jnp.reshape(p0, (BATCH // LANES, LANES))
    p1r = jnp.reshape(p1, (BATCH // LANES, LANES))
    u, v = _phase1(s0, p0r, s1, p1r, emb1.T, emb2.T)
    return _phase2(u, v)


# speedup vs baseline: 3.9847x; 1.1991x over previous
import functools

import jax
import jax.numpy as jnp
from jax import lax
from jax.experimental import pallas as pl
from jax.experimental.pallas import tpu as pltpu
from jax.experimental.pallas import tpu_sc as plsc

VOCAB = 1000000
EMBED = 64
BATCH = 16384
BLK = 128

NUM_CORES = 2
NUM_SUBCORES = 16
LANES = 16
NW = NUM_CORES * NUM_SUBCORES
B_PER_W = BATCH // NW
NQ = EMBED // LANES
DEPTH = 4
LOOKAHEAD = 3


def _make_phase1():
    mesh = plsc.VectorSubcoreMesh(core_axis_name="c", subcore_axis_name="s")

    @functools.partial(
        pl.kernel,
        mesh=mesh,
        out_type=(
            jax.ShapeDtypeStruct((BATCH, BLK), jnp.float32),
            jax.ShapeDtypeStruct((BATCH, BLK), jnp.float32),
        ),
        compiler_params=pltpu.CompilerParams(needs_layout_passes=False),
        scratch_types=[
            pltpu.VMEM((B_PER_W,), jnp.int32),
            pltpu.VMEM((B_PER_W,), jnp.int32),
            pltpu.VMEM((B_PER_W // LANES, LANES), jnp.int32),
            pltpu.VMEM((B_PER_W // LANES, LANES), jnp.int32),
            pltpu.VMEM((DEPTH, EMBED, BLK), jnp.float32),
            pltpu.VMEM((DEPTH, EMBED, BLK), jnp.float32),
            pltpu.VMEM((2, LANES, BLK), jnp.float32),
            pltpu.VMEM((2, LANES, BLK), jnp.float32),
            pltpu.SemaphoreType.DMA,
            pltpu.SemaphoreType.DMA,
            pltpu.SemaphoreType.DMA,
        ],
    )
    def k(s0_hbm, p0_hbm, s1_hbm, p1_hbm, e1t_hbm, e2t_hbm,
          u_hbm, v_hbm, sidx0_v, sidx1_v, pv20, pv21, ring0, ring1,
          rstage0, rstage1, semb0, semb1, sem_s):
        wid = lax.axis_index("s") * NUM_CORES + lax.axis_index("c")
        base = wid * B_PER_W
        lane = lax.iota(jnp.int32, LANES)

        def ext(vec, j):
            return jnp.max(jnp.where(lane == j, vec, 0))

        tabs = []
        for t, (s_hbm, p_hbm, t_hbm, o_hbm, semb, sv, pv, rg, rs) in enumerate((
                (s0_hbm, p0_hbm, e1t_hbm, u_hbm, semb0, sidx0_v, pv20, ring0,
                 rstage0),
                (s1_hbm, p1_hbm, e2t_hbm, v_hbm, semb1, sidx1_v, pv21, ring1,
                 rstage1))):
            pltpu.sync_copy(s_hbm.at[pl.ds(base, B_PER_W)], sv)
            pltpu.sync_copy(
                p_hbm.at[pl.ds(wid * (B_PER_W // LANES), B_PER_W // LANES)],
                pv)
            tabs.append(dict(t=t, t_hbm=t_hbm, o_hbm=o_hbm, semb=semb,
                             sv=sv, pv=pv, ring=rg, rstage=rs))

        def fire(tb, col, slot):
            pltpu.async_copy(
                tb["t_hbm"].at[:, pl.ds(pl.multiple_of(col, BLK), BLK)],
                tb["ring"].at[slot], tb["semb"])

        def drain_block(tb):
            pltpu.make_async_copy(
                tb["t_hbm"].at[:, pl.ds(0, BLK)], tb["ring"].at[0],
                tb["semb"]).wait()

        def drain_scatter(tb):
            pltpu.make_async_copy(
                tb["o_hbm"].at[pl.ds(0, LANES)], tb["rstage"].at[0],
                sem_s).wait()

        jf0s = []
        for tb in tabs:
            sv = tb["sv"]
            cur0 = sv[pl.ds(0, LANES)]
            prv0 = plsc.load_gather(sv, [jnp.maximum(lane - 1, 0)])
            new0 = (((cur0 >> 7) != (prv0 >> 7)) | (lane == 0)) & (
                lane < LOOKAHEAD)
            inc0 = plsc.cumsum(new0.astype(jnp.int32))
            pk0 = (cur0 >> 7) | (inc0 << 13)
            jf_prev = jnp.int32(0)
            for j in range(LOOKAHEAD):
                pj = ext(pk0, j)
                jf_j = pj >> 13

                @pl.when(jf_j != jf_prev)
                def _(pj=pj, jf_j=jf_j, tb=tb):
                    fire(tb, (pj & 8191) << 7, (jf_j - 1) & (DEPTH - 1))

                jf_prev = jf_j
            jf0s.append(jnp.max(inc0))

        def body(g, carry):
            jf0, ju0, jf1, ju1 = carry
            jfs = [jf0, jf1]
            jus = [ju0, ju1]
            gpos = g * LANES + lane
            fpos = gpos + LOOKAHEAD
            fval = fpos < B_PER_W
            fposc = jnp.minimum(fpos, B_PER_W - 1)
            pkas, pkbs = [], []
            for tb in tabs:
                sv = tb["sv"]
                gvec = plsc.load_gather(sv, [gpos])
                prv = plsc.load_gather(sv, [jnp.maximum(gpos - 1, 0)])
                newc = ((gvec >> 7) != (prv >> 7)) | (gpos == 0)
                ju_vec = jus[tb["t"]] + plsc.cumsum(newc.astype(jnp.int32))
                fvec = plsc.load_gather(sv, [fposc])
                fprv = plsc.load_gather(sv, [fposc - 1])
                newf = ((fvec >> 7) != (fprv >> 7)) & fval
                jf_vec = jfs[tb["t"]] + plsc.cumsum(newf.astype(jnp.int32))
                pkas.append((gvec & (BLK - 1)) | (ju_vec << 7))
                pkbs.append((fvec >> 7) | (jf_vec << 13))

            gb = g & 1
            ju_prev = list(jus)
            jf_prev = list(jfs)
            for j in range(LANES):
                for tb in tabs:
                    t = tb["t"]
                    pb = ext(pkbs[t], j)
                    jf_j = pb >> 13

                    @pl.when(jf_j != jf_prev[t])
                    def _(pb=pb, jf_j=jf_j, tb=tb):
                        fire(tb, (pb & 8191) << 7, (jf_j - 1) & (DEPTH - 1))

                    jf_prev[t] = jf_j

                    pa = ext(pkas[t], j)
                    ju_j = pa >> 7

                    @pl.when(ju_j != ju_prev[t])
                    def _(tb=tb):
                        drain_block(tb)

                    ju_prev[t] = ju_j
                    bi = (ju_j - 1) & (DEPTH - 1)
                    cols = jnp.full((LANES,), pa & (BLK - 1), jnp.int32)
                    for q in range(NQ):
                        rows = q * LANES + lane
                        tb["rstage"][gb, j, pl.ds(q * LANES, LANES)] = (
                            plsc.load_gather(tb["ring"].at[bi], [rows, cols]))

            for tb in tabs:
                pltpu.async_copy(
                    tb["rstage"].at[gb], tb["o_hbm"].at[tb["pv"].at[g]], sem_s)

            @pl.when(g >= 1)
            def _():
                drain_scatter(tabs[0])
                drain_scatter(tabs[1])

            return jf_prev[0], ju_prev[0], jf_prev[1], ju_prev[1]

        lax.fori_loop(0, B_PER_W // LANES, body,
                      (jf0s[0], jnp.int32(0), jf0s[1], jnp.int32(0)))
        drain_scatter(tabs[0])
        drain_scatter(tabs[1])

    return k


def _make_phase2():
    mesh = plsc.VectorSubcoreMesh(core_axis_name="c", subcore_axis_name="s")

    @functools.partial(
        pl.kernel,
        mesh=mesh,
        out_type=jax.ShapeDtypeStruct((BATCH,), jnp.float32),
        compiler_params=pltpu.CompilerParams(needs_layout_passes=False),
        scratch_types=[
            pltpu.VMEM((B_PER_W // 2, BLK), jnp.float32),
            pltpu.VMEM((B_PER_W // 2, BLK), jnp.float32),
            pltpu.VMEM((B_PER_W,), jnp.float32),
            pltpu.SemaphoreType.DMA,
            pltpu.SemaphoreType.DMA,
        ],
    )
    def k(u_hbm, v_hbm, out_hbm, u_v, v_v, out_v, sem0, sem1):
        wid = lax.axis_index("s") * NUM_CORES + lax.axis_index("c")
        base = wid * B_PER_W
        half = B_PER_W // 2

        lane = lax.iota(jnp.int32, LANES)
        cols = [jnp.full((LANES,), d, jnp.int32) for d in range(EMBED)]

        for c in range(2):
            cp0 = pltpu.async_copy(
                u_hbm.at[pl.ds(base + c * half, half)], u_v, sem0)
            cp1 = pltpu.async_copy(
                v_hbm.at[pl.ds(base + c * half, half)], v_v, sem1)
            cp0.wait()
            cp1.wait()

            def group(g, carry):
                rows = g * LANES + lane
                acc = jnp.zeros((LANES,), jnp.float32)
                for d in range(EMBED):
                    u = plsc.load_gather(u_v, [rows, cols[d]])
                    v = plsc.load_gather(v_v, [rows, cols[d]])
                    acc = acc + u * v
                out_v[pl.ds(c * half + g * LANES, LANES)] = (
                    1.0 / (1.0 + jnp.exp(-acc)))
                return carry

            lax.fori_loop(0, half // LANES, group, 0)

        pltpu.sync_copy(out_v, out_hbm.at[pl.ds(base, B_PER_W)])

    return k


_phase1 = _make_phase1()
_phase2 = _make_phase2()


@jax.jit
def kernel(X_batch, emb1, emb2):
    idx0 = X_batch[:, 0].astype(jnp.int32)
    idx1 = X_batch[:, 1].astype(jnp.int32)
    pos = lax.iota(jnp.int32, BATCH)
    s0, p0 = lax.sort((idx0, pos), num_keys=1)
    s1, p1 = lax.sort((idx1, pos), num_keys=1)
    p0r = jnp.reshape(p0, (BATCH // LANES, LANES))
    p1r = jnp.reshape(p1, (BATCH // LANES, LANES))
    u, v = _phase1(s0, p0r, s1, p1r, emb1.T, emb2.T)
    return _phase2(u, v)


# speedup vs baseline: 4.4686x; 1.1214x over previous
import functools

import jax
import jax.numpy as jnp
from jax import lax
from jax.experimental import pallas as pl
from jax.experimental.pallas import tpu as pltpu
from jax.experimental.pallas import tpu_sc as plsc

VOCAB = 1000000
EMBED = 64
BATCH = 16384
BLK = 128

NUM_CORES = 2
NUM_SUBCORES = 16
LANES = 16
NW = NUM_CORES * NUM_SUBCORES
B_PER_W = BATCH // NW
NQ = EMBED // LANES
DEPTH = 6
LOOKAHEAD = 5


def _make_phase1():
    mesh = plsc.VectorSubcoreMesh(core_axis_name="c", subcore_axis_name="s")

    @functools.partial(
        pl.kernel,
        mesh=mesh,
        out_type=(
            jax.ShapeDtypeStruct((BATCH, BLK), jnp.float32),
            jax.ShapeDtypeStruct((BATCH, BLK), jnp.float32),
        ),
        compiler_params=pltpu.CompilerParams(needs_layout_passes=False),
        scratch_types=[
            pltpu.VMEM((B_PER_W,), jnp.int32),
            pltpu.VMEM((B_PER_W,), jnp.int32),
            pltpu.VMEM((B_PER_W // LANES, LANES), jnp.int32),
            pltpu.VMEM((B_PER_W // LANES, LANES), jnp.int32),
            pltpu.VMEM((DEPTH, EMBED, BLK), jnp.float32),
            pltpu.VMEM((DEPTH, EMBED, BLK), jnp.float32),
            pltpu.VMEM((2, LANES, BLK), jnp.float32),
            pltpu.VMEM((2, LANES, BLK), jnp.float32),
            pltpu.SemaphoreType.DMA,
            pltpu.SemaphoreType.DMA,
            pltpu.SemaphoreType.DMA,
        ],
    )
    def k(s0_hbm, p0_hbm, s1_hbm, p1_hbm, e1t_hbm, e2t_hbm,
          u_hbm, v_hbm, sidx0_v, sidx1_v, pv20, pv21, ring0, ring1,
          rstage0, rstage1, semb0, semb1, sem_s):
        wid = lax.axis_index("s") * NUM_CORES + lax.axis_index("c")
        base = wid * B_PER_W
        lane = lax.iota(jnp.int32, LANES)

        def ext(vec, j):
            return jnp.max(jnp.where(lane == j, vec, 0))

        tabs = []
        for t, (s_hbm, p_hbm, t_hbm, o_hbm, semb, sv, pv, rg, rs) in enumerate((
                (s0_hbm, p0_hbm, e1t_hbm, u_hbm, semb0, sidx0_v, pv20, ring0,
                 rstage0),
                (s1_hbm, p1_hbm, e2t_hbm, v_hbm, semb1, sidx1_v, pv21, ring1,
                 rstage1))):
            pltpu.sync_copy(s_hbm.at[pl.ds(base, B_PER_W)], sv)
            pltpu.sync_copy(
                p_hbm.at[pl.ds(wid * (B_PER_W // LANES), B_PER_W // LANES)],
                pv)
            tabs.append(dict(t=t, t_hbm=t_hbm, o_hbm=o_hbm, semb=semb,
                             sv=sv, pv=pv, ring=rg, rstage=rs))

        def fire(tb, col, slot):
            pltpu.async_copy(
                tb["t_hbm"].at[:, pl.ds(pl.multiple_of(col, BLK), BLK)],
                tb["ring"].at[slot], tb["semb"])

        def drain_block(tb):
            pltpu.make_async_copy(
                tb["t_hbm"].at[:, pl.ds(0, BLK)], tb["ring"].at[0],
                tb["semb"]).wait()

        def drain_scatter(tb):
            pltpu.make_async_copy(
                tb["o_hbm"].at[pl.ds(0, LANES)], tb["rstage"].at[0],
                sem_s).wait()

        jf0s = []
        for tb in tabs:
            sv = tb["sv"]
            cur0 = sv[pl.ds(0, LANES)]
            prv0 = plsc.load_gather(sv, [jnp.maximum(lane - 1, 0)])
            new0 = (((cur0 >> 7) != (prv0 >> 7)) | (lane == 0)) & (
                lane < LOOKAHEAD)
            inc0 = plsc.cumsum(new0.astype(jnp.int32))
            pk0 = (cur0 >> 7) | (inc0 << 13)
            jf_prev = jnp.int32(0)
            for j in range(LOOKAHEAD):
                pj = ext(pk0, j)
                jf_j = pj >> 13

                @pl.when(jf_j != jf_prev)
                def _(pj=pj, jf_j=jf_j, tb=tb):
                    fire(tb, (pj & 8191) << 7, (jf_j - 1) % DEPTH)

                jf_prev = jf_j
            jf0s.append(jnp.max(inc0))

        def body(g, carry):
            jf0, ju0, jf1, ju1 = carry
            jfs = [jf0, jf1]
            jus = [ju0, ju1]
            gpos = g * LANES + lane
            fpos = gpos + LOOKAHEAD
            fval = fpos < B_PER_W
            fposc = jnp.minimum(fpos, B_PER_W - 1)
            pkas, pkbs = [], []
            for tb in tabs:
                sv = tb["sv"]
                gvec = plsc.load_gather(sv, [gpos])
                prv = plsc.load_gather(sv, [jnp.maximum(gpos - 1, 0)])
                newc = ((gvec >> 7) != (prv >> 7)) | (gpos == 0)
                ju_vec = jus[tb["t"]] + plsc.cumsum(newc.astype(jnp.int32))
                fvec = plsc.load_gather(sv, [fposc])
                fprv = plsc.load_gather(sv, [fposc - 1])
                newf = ((fvec >> 7) != (fprv >> 7)) & fval
                jf_vec = jfs[tb["t"]] + plsc.cumsum(newf.astype(jnp.int32))
                pkas.append((gvec & (BLK - 1)) | (ju_vec << 7))
                pkbs.append((fvec >> 7) | (jf_vec << 13))

            gb = g & 1
            ju_prev = list(jus)
            jf_prev = list(jfs)
            for j in range(LANES):
                for tb in tabs:
                    t = tb["t"]
                    pb = ext(pkbs[t], j)
                    jf_j = pb >> 13

                    @pl.when(jf_j != jf_prev[t])
                    def _(pb=pb, jf_j=jf_j, tb=tb):
                        fire(tb, (pb & 8191) << 7, (jf_j - 1) % DEPTH)

                    jf_prev[t] = jf_j

                    pa = ext(pkas[t], j)
                    ju_j = pa >> 7

                    @pl.when(ju_j != ju_prev[t])
                    def _(tb=tb):
                        drain_block(tb)

                    ju_prev[t] = ju_j
                    bi = (ju_j - 1) % DEPTH
                    cols = jnp.full((LANES,), pa & (BLK - 1), jnp.int32)
                    for q in range(NQ):
                        rows = q * LANES + lane
                        tb["rstage"][gb, j, pl.ds(q * LANES, LANES)] = (
                            plsc.load_gather(tb["ring"].at[bi], [rows, cols]))

            for tb in tabs:
                pltpu.async_copy(
                    tb["rstage"].at[gb], tb["o_hbm"].at[tb["pv"].at[g]], sem_s)

            @pl.when(g >= 1)
            def _():
                drain_scatter(tabs[0])
                drain_scatter(tabs[1])

            return jf_prev[0], ju_prev[0], jf_prev[1], ju_prev[1]

        lax.fori_loop(0, B_PER_W // LANES, body,
                      (jf0s[0], jnp.int32(0), jf0s[1], jnp.int32(0)))
        drain_scatter(tabs[0])
        drain_scatter(tabs[1])

    return k


def _make_phase2():
    mesh = plsc.VectorSubcoreMesh(core_axis_name="c", subcore_axis_name="s")

    @functools.partial(
        pl.kernel,
        mesh=mesh,
        out_type=jax.ShapeDtypeStruct((BATCH,), jnp.float32),
        compiler_params=pltpu.CompilerParams(needs_layout_passes=False),
        scratch_types=[
            pltpu.VMEM((B_PER_W // 2, BLK), jnp.float32),
            pltpu.VMEM((B_PER_W // 2, BLK), jnp.float32),
            pltpu.VMEM((B_PER_W,), jnp.float32),
            pltpu.SemaphoreType.DMA,
            pltpu.SemaphoreType.DMA,
        ],
    )
    def k(u_hbm, v_hbm, out_hbm, u_v, v_v, out_v, sem0, sem1):
        wid = lax.axis_index("s") * NUM_CORES + lax.axis_index("c")
        base = wid * B_PER_W
        half = B_PER_W // 2

        lane = lax.iota(jnp.int32, LANES)
        cols = [jnp.full((LANES,), d, jnp.int32) for d in range(EMBED)]

        for c in range(2):
            cp0 = pltpu.async_copy(
                u_hbm.at[pl.ds(base + c * half, half)], u_v, sem0)
            cp1 = pltpu.async_copy(
                v_hbm.at[pl.ds(base + c * half, half)], v_v, sem1)
            cp0.wait()
            cp1.wait()

            def group(g, carry):
                rows = g * LANES + lane
                acc = jnp.zeros((LANES,), jnp.float32)
                for d in range(EMBED):
                    u = plsc.load_gather(u_v, [rows, cols[d]])
                    v = plsc.load_gather(v_v, [rows, cols[d]])
                    acc = acc + u * v
                out_v[pl.ds(c * half + g * LANES, LANES)] = (
                    1.0 / (1.0 + jnp.exp(-acc)))
                return carry

            lax.fori_loop(0, half // LANES, group, 0)

        pltpu.sync_copy(out_v, out_hbm.at[pl.ds(base, B_PER_W)])

    return k


_phase1 = _make_phase1()
_phase2 = _make_phase2()


@jax.jit
def kernel(X_batch, emb1, emb2):
    idx0 = X_batch[:, 0].astype(jnp.int32)
    idx1 = X_batch[:, 1].astype(jnp.int32)
    pos = lax.iota(jnp.int32, BATCH)
    s0, p0 = lax.sort((idx0, pos), num_keys=1)
    s1, p1 = lax.sort((idx1, pos), num_keys=1)
    p0r = jnp.reshape(p0, (BATCH // LANES, LANES))
    p1r = jnp.reshape(p1, (BATCH // LANES, LANES))
    u, v = _phase1(s0, p0r, s1, p1r, emb1.T, emb2.T)
    return _phase2(u, v)


# speedup vs baseline: 5.1154x; 1.1447x over previous
import functools

import jax
import jax.numpy as jnp
from jax import lax
from jax.experimental import pallas as pl
from jax.experimental.pallas import tpu as pltpu
from jax.experimental.pallas import tpu_sc as plsc

VOCAB = 1000000
EMBED = 64
BATCH = 16384
BLK = 128

NUM_CORES = 2
NUM_SUBCORES = 16
LANES = 16
NW = NUM_CORES * NUM_SUBCORES
B_PER_W = BATCH // NW
NQ = EMBED // LANES
DEPTH = 6
LOOKAHEAD = 5


def _make_phase1():
    mesh = plsc.VectorSubcoreMesh(core_axis_name="c", subcore_axis_name="s")

    @functools.partial(
        pl.kernel,
        mesh=mesh,
        out_type=(
            jax.ShapeDtypeStruct((BATCH, BLK), jnp.float32),
            jax.ShapeDtypeStruct((BATCH, BLK), jnp.float32),
        ),
        compiler_params=pltpu.CompilerParams(needs_layout_passes=False),
        scratch_types=[
            pltpu.VMEM((B_PER_W,), jnp.int32),
            pltpu.VMEM((B_PER_W,), jnp.int32),
            pltpu.VMEM((B_PER_W // LANES, LANES), jnp.int32),
            pltpu.VMEM((B_PER_W // LANES, LANES), jnp.int32),
            pltpu.VMEM((DEPTH, EMBED, BLK), jnp.float32),
            pltpu.VMEM((DEPTH, EMBED, BLK), jnp.float32),
            pltpu.VMEM((2, LANES, BLK), jnp.float32),
            pltpu.VMEM((2, LANES, BLK), jnp.float32),
            pltpu.SemaphoreType.DMA,
            pltpu.SemaphoreType.DMA,
            pltpu.SemaphoreType.DMA,
        ],
    )
    def k(s0_hbm, p0_hbm, s1_hbm, p1_hbm, e1t_hbm, e2t_hbm,
          u_hbm, v_hbm, sidx0_v, sidx1_v, pv20, pv21, ring0, ring1,
          rstage0, rstage1, semb0, semb1, sem_s):
        wid = lax.axis_index("s") * NUM_CORES + lax.axis_index("c")
        base = wid * B_PER_W
        lane = lax.iota(jnp.int32, LANES)

        def ext(vec, j):
            return jnp.max(jnp.where(lane == j, vec, 0))

        tabs = []
        for t, (s_hbm, p_hbm, t_hbm, o_hbm, semb, sv, pv, rg, rs) in enumerate((
                (s0_hbm, p0_hbm, e1t_hbm, u_hbm, semb0, sidx0_v, pv20, ring0,
                 rstage0),
                (s1_hbm, p1_hbm, e2t_hbm, v_hbm, semb1, sidx1_v, pv21, ring1,
                 rstage1))):
            pltpu.sync_copy(s_hbm.at[pl.ds(base, B_PER_W)], sv)
            pltpu.sync_copy(
                p_hbm.at[pl.ds(wid * (B_PER_W // LANES), B_PER_W // LANES)],
                pv)
            tabs.append(dict(t=t, t_hbm=t_hbm, o_hbm=o_hbm, semb=semb,
                             sv=sv, pv=pv, ring=rg, rstage=rs))

        def fire(tb, col, slot):
            pltpu.async_copy(
                tb["t_hbm"].at[:, pl.ds(pl.multiple_of(col, BLK), BLK)],
                tb["ring"].at[slot], tb["semb"])

        def drain_block(tb):
            pltpu.make_async_copy(
                tb["t_hbm"].at[:, pl.ds(0, BLK)], tb["ring"].at[0],
                tb["semb"]).wait()

        def drain_scatter(tb):
            pltpu.make_async_copy(
                tb["o_hbm"].at[pl.ds(0, LANES)], tb["rstage"].at[0],
                sem_s).wait()

        jf0s = []
        for tb in tabs:
            sv = tb["sv"]
            cur0 = sv[pl.ds(0, LANES)]
            prv0 = plsc.load_gather(sv, [jnp.maximum(lane - 1, 0)])
            new0 = (((cur0 >> 7) != (prv0 >> 7)) | (lane == 0)) & (
                lane < LOOKAHEAD)
            inc0 = plsc.cumsum(new0.astype(jnp.int32))
            pk0 = (cur0 >> 7) | (inc0 << 13)
            jf_prev = jnp.int32(0)
            for j in range(LOOKAHEAD):
                pj = ext(pk0, j)
                jf_j = pj >> 13

                @pl.when(jf_j != jf_prev)
                def _(pj=pj, jf_j=jf_j, tb=tb):
                    fire(tb, (pj & 8191) << 7, (jf_j - 1) % DEPTH)

                jf_prev = jf_j
            jf0s.append(jnp.max(inc0))

        def body(g, carry):
            jf0, ju0, jf1, ju1 = carry
            jfs = [jf0, jf1]
            jus = [ju0, ju1]
            gpos = g * LANES + lane
            fpos = gpos + LOOKAHEAD
            fval = fpos < B_PER_W
            fposc = jnp.minimum(fpos, B_PER_W - 1)
            pkas, pkbs = [], []
            for tb in tabs:
                sv = tb["sv"]
                gvec = plsc.load_gather(sv, [gpos])
                prv = plsc.load_gather(sv, [jnp.maximum(gpos - 1, 0)])
                newc = ((gvec >> 7) != (prv >> 7)) | (gpos == 0)
                ju_vec = jus[tb["t"]] + plsc.cumsum(newc.astype(jnp.int32))
                fvec = plsc.load_gather(sv, [fposc])
                fprv = plsc.load_gather(sv, [fposc - 1])
                newf = ((fvec >> 7) != (fprv >> 7)) & fval
                jf_vec = jfs[tb["t"]] + plsc.cumsum(newf.astype(jnp.int32))
                pkas.append((gvec & (BLK - 1)) | (ju_vec << 7))
                pkbs.append((fvec >> 7) | (jf_vec << 13))

            gb = g & 1
            ju_prev = list(jus)
            jf_prev = list(jfs)
            for j in range(LANES):
                for tb in tabs:
                    t = tb["t"]
                    pb = ext(pkbs[t], j)
                    jf_j = pb >> 13

                    @pl.when(jf_j != jf_prev[t])
                    def _(pb=pb, jf_j=jf_j, tb=tb):
                        fire(tb, (pb & 8191) << 7, (jf_j - 1) % DEPTH)

                    jf_prev[t] = jf_j

                    pa = ext(pkas[t], j)
                    ju_j = pa >> 7

                    @pl.when(ju_j != ju_prev[t])
                    def _(tb=tb):
                        drain_block(tb)

                    ju_prev[t] = ju_j
                    bi = (ju_j - 1) % DEPTH
                    cols = jnp.full((LANES,), pa & (BLK - 1), jnp.int32)
                    for q in range(NQ):
                        rows = q * LANES + lane
                        tb["rstage"][gb, j, pl.ds(q * LANES, LANES)] = (
                            plsc.load_gather(tb["ring"].at[bi], [rows, cols]))

            for tb in tabs:
                pltpu.async_copy(
                    tb["rstage"].at[gb], tb["o_hbm"].at[tb["pv"].at[g]], sem_s)

            @pl.when(g >= 1)
            def _():
                drain_scatter(tabs[0])
                drain_scatter(tabs[1])

            return jf_prev[0], ju_prev[0], jf_prev[1], ju_prev[1]

        lax.fori_loop(0, B_PER_W // LANES, body,
                      (jf0s[0], jnp.int32(0), jf0s[1], jnp.int32(0)))
        drain_scatter(tabs[0])
        drain_scatter(tabs[1])

    return k


def _make_phase2():
    rows = 2048

    def body(u_ref, v_ref, o_ref):
        w = u_ref[:, :EMBED] * v_ref[:, :EMBED]
        o_ref[...] = 1.0 / (1.0 + jnp.exp(-jnp.sum(w, axis=1)))

    return pl.pallas_call(
        body,
        grid=(BATCH // rows,),
        in_specs=[
            pl.BlockSpec((rows, BLK), lambda i: (i, 0)),
            pl.BlockSpec((rows, BLK), lambda i: (i, 0)),
        ],
        out_specs=pl.BlockSpec((rows,), lambda i: (i,)),
        out_shape=jax.ShapeDtypeStruct((BATCH,), jnp.float32),
    )


_phase1 = _make_phase1()
_phase2 = _make_phase2()


@jax.jit
def kernel(X_batch, emb1, emb2):
    idx0 = X_batch[:, 0].astype(jnp.int32)
    idx1 = X_batch[:, 1].astype(jnp.int32)
    pos = lax.iota(jnp.int32, BATCH)
    s0, p0 = lax.sort((idx0, pos), num_keys=1)
    s1, p1 = lax.sort((idx1, pos), num_keys=1)
    p0r = jnp.reshape(p0, (BATCH // LANES, LANES))
    p1r = jnp.reshape(p1, (BATCH // LANES, LANES))
    u, v = _phase1(s0, p0r, s1, p1r, emb1.T, emb2.T)
    return _phase2(u, v)


# speedup vs baseline: 5.1248x; 1.0018x over previous
import functools

import jax
import jax.numpy as jnp
from jax import lax
from jax.experimental import pallas as pl
from jax.experimental.pallas import tpu as pltpu
from jax.experimental.pallas import tpu_sc as plsc

VOCAB = 1000000
EMBED = 64
BATCH = 16384
BLK = 128

NUM_CORES = 2
NUM_SUBCORES = 16
LANES = 16
NW = NUM_CORES * NUM_SUBCORES
B_PER_W = BATCH // NW
NQ = EMBED // LANES
DEPTH = 6
LOOKAHEAD = 5


def _make_phase1():
    mesh = plsc.VectorSubcoreMesh(core_axis_name="c", subcore_axis_name="s")

    @functools.partial(
        pl.kernel,
        mesh=mesh,
        out_type=(
            jax.ShapeDtypeStruct((BATCH, BLK), jnp.float32),
            jax.ShapeDtypeStruct((BATCH, BLK), jnp.float32),
        ),
        compiler_params=pltpu.CompilerParams(needs_layout_passes=False),
        scratch_types=[
            pltpu.VMEM((B_PER_W,), jnp.int32),
            pltpu.VMEM((B_PER_W,), jnp.int32),
            pltpu.VMEM((B_PER_W // LANES, LANES), jnp.int32),
            pltpu.VMEM((B_PER_W // LANES, LANES), jnp.int32),
            pltpu.VMEM((DEPTH, EMBED, BLK), jnp.float32),
            pltpu.VMEM((DEPTH, EMBED, BLK), jnp.float32),
            pltpu.VMEM((2, LANES, BLK), jnp.float32),
            pltpu.VMEM((2, LANES, BLK), jnp.float32),
            pltpu.SemaphoreType.DMA,
            pltpu.SemaphoreType.DMA,
            pltpu.SemaphoreType.DMA,
        ],
    )
    def k(s0_hbm, p0_hbm, s1_hbm, p1_hbm, e1t_hbm, e2t_hbm,
          u_hbm, v_hbm, sidx0_v, sidx1_v, pv20, pv21, ring0, ring1,
          rstage0, rstage1, semb0, semb1, sem_s):
        wid = lax.axis_index("s") * NUM_CORES + lax.axis_index("c")
        base = wid * B_PER_W
        lane = lax.iota(jnp.int32, LANES)

        def ext(vec, j):
            return jnp.max(jnp.where(lane == j, vec, 0))

        tabs = []
        for t, (s_hbm, p_hbm, t_hbm, o_hbm, semb, sv, pv, rg, rs) in enumerate((
                (s0_hbm, p0_hbm, e1t_hbm, u_hbm, semb0, sidx0_v, pv20, ring0,
                 rstage0),
                (s1_hbm, p1_hbm, e2t_hbm, v_hbm, semb1, sidx1_v, pv21, ring1,
                 rstage1))):
            pltpu.sync_copy(s_hbm.at[pl.ds(base, B_PER_W)], sv)
            pltpu.sync_copy(
                p_hbm.at[pl.ds(wid * (B_PER_W // LANES), B_PER_W // LANES)],
                pv)
            tabs.append(dict(t=t, t_hbm=t_hbm, o_hbm=o_hbm, semb=semb,
                             sv=sv, pv=pv, ring=rg, rstage=rs))

        def fire(tb, col, slot):
            pltpu.async_copy(
                tb["t_hbm"].at[:, pl.ds(pl.multiple_of(col, BLK), BLK)],
                tb["ring"].at[slot], tb["semb"])

        def drain_block(tb):
            pltpu.make_async_copy(
                tb["t_hbm"].at[:, pl.ds(0, BLK)], tb["ring"].at[0],
                tb["semb"]).wait()

        def drain_scatter(tb):
            pltpu.make_async_copy(
                tb["o_hbm"].at[pl.ds(0, LANES)], tb["rstage"].at[0],
                sem_s).wait()

        jf0s = []
        for tb in tabs:
            sv = tb["sv"]
            cur0 = sv[pl.ds(0, LANES)]
            prv0 = plsc.load_gather(sv, [jnp.maximum(lane - 1, 0)])
            new0 = (((cur0 >> 7) != (prv0 >> 7)) | (lane == 0)) & (
                lane < LOOKAHEAD)
            inc0 = plsc.cumsum(new0.astype(jnp.int32))
            pk0 = (cur0 >> 7) | (inc0 << 13)
            jf_prev = jnp.int32(0)
            for j in range(LOOKAHEAD):
                pj = ext(pk0, j)
                jf_j = pj >> 13

                @pl.when(jf_j != jf_prev)
                def _(pj=pj, jf_j=jf_j, tb=tb):
                    fire(tb, (pj & 8191) << 7, (jf_j - 1) % DEPTH)

                jf_prev = jf_j
            jf0s.append(jnp.max(inc0))

        def body(g, carry):
            jf0, ju0, jf1, ju1 = carry
            jfs = [jf0, jf1]
            jus = [ju0, ju1]
            gpos = g * LANES + lane
            fpos = gpos + LOOKAHEAD
            fval = fpos < B_PER_W
            fposc = jnp.minimum(fpos, B_PER_W - 1)
            pks = []
            for tb in tabs:
                sv = tb["sv"]
                gvec = plsc.load_gather(sv, [gpos])
                prv = plsc.load_gather(sv, [jnp.maximum(gpos - 1, 0)])
                newc = ((gvec >> 7) != (prv >> 7)) | (gpos == 0)
                dju = plsc.cumsum(newc.astype(jnp.int32))
                fvec = plsc.load_gather(sv, [fposc])
                fprv = plsc.load_gather(sv, [fposc - 1])
                newf = ((fvec >> 7) != (fprv >> 7)) & fval
                djf = plsc.cumsum(newf.astype(jnp.int32))
                pks.append((gvec & (BLK - 1)) | (dju << 7) | (djf << 12)
                           | ((fvec >> 7) << 17))

            gb = g & 1
            ju_prev = list(jus)
            jf_prev = list(jfs)
            for j in range(LANES):
                for tb in tabs:
                    t = tb["t"]
                    pa = ext(pks[t], j)
                    jf_j = jfs[t] + ((pa >> 12) & 31)

                    @pl.when(jf_j != jf_prev[t])
                    def _(pa=pa, jf_j=jf_j, tb=tb):
                        fire(tb, (pa >> 17) << 7, (jf_j - 1) % DEPTH)

                    jf_prev[t] = jf_j
                    ju_j = jus[t] + ((pa >> 7) & 31)

                    @pl.when(ju_j != ju_prev[t])
                    def _(tb=tb):
                        drain_block(tb)

                    ju_prev[t] = ju_j
                    bi = (ju_j - 1) % DEPTH
                    cols = jnp.full((LANES,), pa & (BLK - 1), jnp.int32)
                    for q in range(NQ):
                        rows = q * LANES + lane
                        tb["rstage"][gb, j, pl.ds(q * LANES, LANES)] = (
                            plsc.load_gather(tb["ring"].at[bi], [rows, cols]))

            for tb in tabs:
                pltpu.async_copy(
                    tb["rstage"].at[gb], tb["o_hbm"].at[tb["pv"].at[g]], sem_s)

            @pl.when(g >= 1)
            def _():
                drain_scatter(tabs[0])
                drain_scatter(tabs[1])

            return jf_prev[0], ju_prev[0], jf_prev[1], ju_prev[1]

        lax.fori_loop(0, B_PER_W // LANES, body,
                      (jf0s[0], jnp.int32(0), jf0s[1], jnp.int32(0)))
        drain_scatter(tabs[0])
        drain_scatter(tabs[1])

    return k


def _make_phase2():
    rows = 2048

    def body(u_ref, v_ref, o_ref):
        w = u_ref[:, :EMBED] * v_ref[:, :EMBED]
        o_ref[...] = 1.0 / (1.0 + jnp.exp(-jnp.sum(w, axis=1)))

    return pl.pallas_call(
        body,
        grid=(BATCH // rows,),
        in_specs=[
            pl.BlockSpec((rows, BLK), lambda i: (i, 0)),
            pl.BlockSpec((rows, BLK), lambda i: (i, 0)),
        ],
        out_specs=pl.BlockSpec((rows,), lambda i: (i,)),
        out_shape=jax.ShapeDtypeStruct((BATCH,), jnp.float32),
    )


_phase1 = _make_phase1()
_phase2 = _make_phase2()


@jax.jit
def kernel(X_batch, emb1, emb2):
    idx0 = X_batch[:, 0].astype(jnp.int32)
    idx1 = X_batch[:, 1].astype(jnp.int32)
    pos = lax.iota(jnp.int32, BATCH)
    s0, p0 = lax.sort((idx0, pos), num_keys=1)
    s1, p1 = lax.sort((idx1, pos), num_keys=1)
    p0r = jnp.reshape(p0, (BATCH // LANES, LANES))
    p1r = jnp.reshape(p1, (BATCH // LANES, LANES))
    u, v = _phase1(s0, p0r, s1, p1r, emb1.T, emb2.T)
    return _phase2(u, v)


# speedup vs baseline: 5.1268x; 1.0004x over previous
import functools

import jax
import jax.numpy as jnp
from jax import lax
from jax.experimental import pallas as pl
from jax.experimental.pallas import tpu as pltpu
from jax.experimental.pallas import tpu_sc as plsc

VOCAB = 1000000
EMBED = 64
BATCH = 16384
BLK = 128

NUM_CORES = 2
NUM_SUBCORES = 16
LANES = 16
NW = NUM_CORES * NUM_SUBCORES
B_PER_W = BATCH // NW
NQ = EMBED // LANES
DEPTH = 6
LOOKAHEAD = 5


def _make_phase1():
    mesh = plsc.VectorSubcoreMesh(core_axis_name="c", subcore_axis_name="s")

    @functools.partial(
        pl.kernel,
        mesh=mesh,
        out_type=(
            jax.ShapeDtypeStruct((BATCH, BLK), jnp.float32),
            jax.ShapeDtypeStruct((BATCH, BLK), jnp.float32),
        ),
        compiler_params=pltpu.CompilerParams(needs_layout_passes=False),
        scratch_types=[
            pltpu.VMEM((B_PER_W,), jnp.int32),
            pltpu.VMEM((B_PER_W,), jnp.int32),
            pltpu.VMEM((B_PER_W // LANES, LANES), jnp.int32),
            pltpu.VMEM((B_PER_W // LANES, LANES), jnp.int32),
            pltpu.VMEM((DEPTH, EMBED, BLK), jnp.float32),
            pltpu.VMEM((DEPTH, EMBED, BLK), jnp.float32),
            pltpu.VMEM((2, LANES, BLK), jnp.float32),
            pltpu.VMEM((2, LANES, BLK), jnp.float32),
            pltpu.SemaphoreType.DMA,
            pltpu.SemaphoreType.DMA,
            pltpu.SemaphoreType.DMA,
        ],
    )
    def k(s0_hbm, p0_hbm, s1_hbm, p1_hbm, e1t_hbm, e2t_hbm,
          u_hbm, v_hbm, sidx0_v, sidx1_v, pv20, pv21, ring0, ring1,
          rstage0, rstage1, semb0, semb1, sem_s):
        wid = lax.axis_index("s") * NUM_CORES + lax.axis_index("c")
        base = wid * B_PER_W
        lane = lax.iota(jnp.int32, LANES)

        def ext(vec, j):
            return jnp.max(jnp.where(lane == j, vec, 0))

        tabs = []
        for t, (s_hbm, p_hbm, t_hbm, o_hbm, semb, sv, pv, rg, rs) in enumerate((
                (s0_hbm, p0_hbm, e1t_hbm, u_hbm, semb0, sidx0_v, pv20, ring0,
                 rstage0),
                (s1_hbm, p1_hbm, e2t_hbm, v_hbm, semb1, sidx1_v, pv21, ring1,
                 rstage1))):
            pltpu.sync_copy(s_hbm.at[pl.ds(base, B_PER_W)], sv)
            pltpu.sync_copy(
                p_hbm.at[pl.ds(wid * (B_PER_W // LANES), B_PER_W // LANES)],
                pv)
            tabs.append(dict(t=t, t_hbm=t_hbm, o_hbm=o_hbm, semb=semb,
                             sv=sv, pv=pv, ring=rg, rstage=rs))

        def fire(tb, col, slot):
            pltpu.async_copy(
                tb["t_hbm"].at[:, pl.ds(pl.multiple_of(col, BLK), BLK)],
                tb["ring"].at[slot], tb["semb"])

        def drain_block(tb):
            pltpu.make_async_copy(
                tb["t_hbm"].at[:, pl.ds(0, BLK)], tb["ring"].at[0],
                tb["semb"]).wait()

        def drain_scatter(tb):
            pltpu.make_async_copy(
                tb["o_hbm"].at[pl.ds(0, LANES)], tb["rstage"].at[0],
                sem_s).wait()

        jf0s = []
        for tb in tabs:
            sv = tb["sv"]
            cur0 = sv[pl.ds(0, LANES)]
            prv0 = plsc.load_gather(sv, [jnp.maximum(lane - 1, 0)])
            new0 = (((cur0 >> 7) != (prv0 >> 7)) | (lane == 0)) & (
                lane < LOOKAHEAD)
            inc0 = plsc.cumsum(new0.astype(jnp.int32))
            pk0 = (cur0 >> 7) | (inc0 << 13)
            jf_prev = jnp.int32(0)
            for j in range(LOOKAHEAD):
                pj = ext(pk0, j)
                jf_j = pj >> 13

                @pl.when(jf_j != jf_prev)
                def _(pj=pj, jf_j=jf_j, tb=tb):
                    fire(tb, (pj & 8191) << 7, (jf_j - 1) % DEPTH)

                jf_prev = jf_j
            jf0s.append(jnp.max(inc0))

        def body(g, carry):
            jf0, ju0, jf1, ju1 = carry
            jfs = [jf0, jf1]
            jus = [ju0, ju1]
            gpos = g * LANES + lane
            fpos = gpos + LOOKAHEAD
            fval = fpos < B_PER_W
            fposc = jnp.minimum(fpos, B_PER_W - 1)
            pks = []
            for tb in tabs:
                sv = tb["sv"]
                gvec = plsc.load_gather(sv, [gpos])
                prv = plsc.load_gather(sv, [jnp.maximum(gpos - 1, 0)])
                newc = ((gvec >> 7) != (prv >> 7)) | (gpos == 0)
                dju = plsc.cumsum(newc.astype(jnp.int32))
                fvec = plsc.load_gather(sv, [fposc])
                fprv = plsc.load_gather(sv, [fposc - 1])
                newf = ((fvec >> 7) != (fprv >> 7)) & fval
                djf = plsc.cumsum(newf.astype(jnp.int32))
                pks.append((gvec & (BLK - 1)) | (dju << 7) | (djf << 12)
                           | ((fvec >> 7) << 17))

            gb = g & 1
            ju_prev = list(jus)
            jf_prev = list(jfs)
            for j in range(LANES):
                for tb in tabs:
                    t = tb["t"]
                    pa = ext(pks[t], j)
                    jf_j = jfs[t] + ((pa >> 12) & 31)

                    @pl.when(jf_j != jf_prev[t])
                    def _(pa=pa, jf_j=jf_j, tb=tb):
                        fire(tb, (pa >> 17) << 7, (jf_j - 1) % DEPTH)

                    jf_prev[t] = jf_j
                    ju_j = jus[t] + ((pa >> 7) & 31)

                    @pl.when(ju_j != ju_prev[t])
                    def _(tb=tb):
                        drain_block(tb)

                    ju_prev[t] = ju_j
                    bi = (ju_j - 1) % DEPTH
                    cols = jnp.full((LANES,), pa & (BLK - 1), jnp.int32)
                    for q in range(NQ):
                        rows = q * LANES + lane
                        tb["rstage"][gb, j, pl.ds(q * LANES, LANES)] = (
                            plsc.load_gather(tb["ring"].at[bi], [rows, cols]))

            for tb in tabs:
                pltpu.async_copy(
                    tb["rstage"].at[gb], tb["o_hbm"].at[tb["pv"].at[g]], sem_s)

            @pl.when(g >= 1)
            def _():
                drain_scatter(tabs[0])
                drain_scatter(tabs[1])

            return jf_prev[0], ju_prev[0], jf_prev[1], ju_prev[1]

        lax.fori_loop(0, B_PER_W // LANES, body,
                      (jf0s[0], jnp.int32(0), jf0s[1], jnp.int32(0)))
        drain_scatter(tabs[0])
        drain_scatter(tabs[1])

    return k


def _make_phase2():
    rows = 8192

    def body(u_ref, v_ref, o_ref):
        w = u_ref[:, :EMBED] * v_ref[:, :EMBED]
        o_ref[...] = 1.0 / (1.0 + jnp.exp(-jnp.sum(w, axis=1)))

    return pl.pallas_call(
        body,
        grid=(BATCH // rows,),
        in_specs=[
            pl.BlockSpec((rows, BLK), lambda i: (i, 0)),
            pl.BlockSpec((rows, BLK), lambda i: (i, 0)),
        ],
        out_specs=pl.BlockSpec((rows,), lambda i: (i,)),
        out_shape=jax.ShapeDtypeStruct((BATCH,), jnp.float32),
    )


_phase1 = _make_phase1()
_phase2 = _make_phase2()


@jax.jit
def kernel(X_batch, emb1, emb2):
    idx0 = X_batch[:, 0].astype(jnp.int32)
    idx1 = X_batch[:, 1].astype(jnp.int32)
    pos = lax.iota(jnp.int32, BATCH)
    s0, p0 = lax.sort((idx0, pos), num_keys=1)
    s1, p1 = lax.sort((idx1, pos), num_keys=1)
    p0r = jnp.reshape(p0, (BATCH // LANES, LANES))
    p1r = jnp.reshape(p1, (BATCH // LANES, LANES))
    u, v = _phase1(s0, p0r, s1, p1r, emb1.T, emb2.T)
    return _phase2(u, v)
